# baseline (speedup 1.0000x reference)
import functools

import jax
import jax.numpy as jnp
from jax import lax
from jax.experimental import pallas as pl
from jax.experimental.pallas import tpu as pltpu
from jax.experimental.pallas import tpu_sc as plsc

_TOPK = 1024
_ROWS = 64
_COLS = 32768
_TC_BLOCK = 8


def _sc_body(x_hbm, out_hbm, xv):
    c = lax.axis_index("c")
    s = lax.axis_index("s")
    wid = s * 2 + c
    base = wid * 16
    pltpu.sync_copy(x_hbm.at[pl.ds(base, 16)], xv)
    pltpu.sync_copy(xv, out_hbm.at[pl.ds(base, 16)])


_sc_kernel = functools.partial(
    pl.kernel,
    out_type=jax.ShapeDtypeStruct((512,), jnp.float32),
    mesh=plsc.VectorSubcoreMesh(core_axis_name="c", subcore_axis_name="s"),
    scratch_types=[
        pltpu.VMEM((16,), jnp.float32),
    ],
    compiler_params=pltpu.CompilerParams(needs_layout_passes=False),
)(_sc_body)


def _tc_body(x_ref, o_ref):
    x = x_ref[...]
    u = lax.bitcast_convert_type(x, jnp.uint32)
    sign = u >> jnp.uint32(31)
    key = u ^ (jnp.uint32(0x80000000) + sign * jnp.uint32(0x7FFFFFFF))

    def step(i, p):
        bit = jnp.uint32(31) - i.astype(jnp.uint32)
        cand = p | (jnp.uint32(1) << bit)
        cnt = jnp.sum((key >= cand).astype(jnp.int32), axis=1, keepdims=True)
        return jnp.where(cnt >= _TOPK, cand, p)

    p0 = jnp.zeros((x.shape[0], 1), jnp.uint32)
    thresh = lax.fori_loop(0, 32, step, p0)
    o_ref[...] = jnp.where(key >= thresh, x, jnp.float32(0.0))


def _tc_kernel(x):
    return pl.pallas_call(
        _tc_body,
        out_shape=jax.ShapeDtypeStruct((_ROWS, _COLS), jnp.float32),
        grid=(_ROWS // _TC_BLOCK,),
        in_specs=[pl.BlockSpec((_TC_BLOCK, _COLS), lambda i: (i, 0))],
        out_specs=pl.BlockSpec((_TC_BLOCK, _COLS), lambda i: (i, 0)),
    )(x)


def kernel(x):
    probe = _sc_kernel(x.reshape(-1)[:512])
    out = _tc_kernel(x)
    return out + 0.0 * probe[0]


# speedup vs baseline: 1.4411x; 1.4411x over previous
import functools

import jax
import jax.numpy as jnp
from jax import lax
from jax.experimental import pallas as pl
from jax.experimental.pallas import tpu as pltpu
from jax.experimental.pallas import tpu_sc as plsc

_TOPK = 1024
_ROWS = 64
_COLS = 32768
_NW = 32
_NV = _COLS // 16
_UNROLL = 8

_SC_ROWS = 32
_TC_ROWS = _ROWS - _SC_ROWS
_TC_BLOCK = 8


def _find_bucket(hist, tot, k):
    lane = lax.iota(jnp.int32, 16)

    @plsc.parallel_loop(0, 16, unroll=4)
    def _(c):
        acc = jnp.zeros((16,), jnp.int32)
        for l in range(16):
            acc = acc + hist[pl.ds(l * 256 + c * 16, 16)]
        tot[pl.ds(c * 16, 16)] = acc

    def cond(st):
        return (st[0] >= 0) & jnp.logical_not(st[1])

    def body(st):
        c, _, acc, _, _, _ = st
        tv = tot[pl.ds(c * 16, 16)]
        rv = lax.rev(tv, (0,))
        cs = plsc.cumsum(rv) + acc
        hit = cs >= k
        nhit = plsc.cumsum(hit.astype(jnp.int32))
        first = hit & (nhit == 1)
        s_b = jnp.sum(jnp.where(first, cs, 0))
        c_b = jnp.sum(jnp.where(first, rv, 0))
        l_b = jnp.sum(jnp.where(first, lane, 0))
        found = jnp.sum(hit.astype(jnp.int32)) > 0
        chunk_total = jnp.sum(tv)
        bucket = jnp.where(found, c * 16 + 15 - l_b, 0)
        rank = jnp.where(found, k - (s_b - c_b), k)
        return (c - 1, found, acc + chunk_total, bucket, rank, c_b)

    _, _, _, bucket, rank, cnt = lax.while_loop(
        cond,
        body,
        (
            jnp.int32(15),
            jnp.bool_(False),
            jnp.int32(0),
            jnp.int32(0),
            k,
            jnp.int32(0),
        ),
    )
    return bucket, rank, cnt


def _sc_body(x_hbm, out_hbm, xv, kv, cand, hist, tot):
    c = lax.axis_index("c")
    s = lax.axis_index("s")
    wid = s * 2 + c
    lane = lax.iota(jnp.int32, 16)
    lane256 = lane * 256
    ones = jnp.ones((16,), jnp.int32)
    zeros16 = jnp.zeros((16,), jnp.int32)

    def zero_hist():
        @plsc.parallel_loop(0, 256, unroll=_UNROLL)
        def _(i):
            hist[pl.ds(i * 16, 16)] = zeros16

    base = wid * _COLS
    pltpu.sync_copy(x_hbm.at[pl.ds(base, _COLS)], xv)

    zero_hist()

    @plsc.parallel_loop(0, _NV, unroll=_UNROLL)
    def _(i):
        xc = xv[pl.ds(i * 16, 16)]
        u = lax.bitcast_convert_type(xc, jnp.uint32)
        sign = u >> jnp.uint32(31)
        key = u ^ (jnp.uint32(0x80000000) + sign * jnp.uint32(0x7FFFFFFF))
        kv[pl.ds(i * 16, 16)] = key
        b = (key >> jnp.uint32(24)).astype(jnp.int32)
        plsc.addupdate_scatter(hist, [lane256 + b], ones)

    b1, k, cnt1 = _find_bucket(hist, tot, jnp.int32(_TOPK))
    b1u = b1.astype(jnp.uint32)
    zero_hist()

    @plsc.parallel_loop(
        0, _NV, unroll=_UNROLL, carry=jnp.zeros((16,), jnp.int32)
    )
    def _(i, ofs):
        key = kv[pl.ds(i * 16, 16)]
        elig = (key >> jnp.uint32(24)) == b1u
        b = ((key >> jnp.uint32(16)) & jnp.uint32(0xFF)).astype(jnp.int32)
        plsc.addupdate_scatter(hist, [lane256 + b], ones, mask=elig)
        pos = plsc.cumsum(elig.astype(jnp.int32))
        plsc.store_scatter(
            cand,
            [ofs + pos - 1],
            lax.bitcast_convert_type(key, jnp.int32),
            mask=elig,
        )
        return ofs + plsc.all_reduce_population_count(elig)

    b2, k, _ = _find_bucket(hist, tot, k)
    b2u = b2.astype(jnp.uint32)
    nv1 = (cnt1 + 15) >> 4
    zero_hist()

    @plsc.parallel_loop(0, nv1, unroll=4)
    def _(i):
        kc = lax.bitcast_convert_type(cand[pl.ds(i * 16, 16)], jnp.uint32)
        valid = (i * 16 + lane) < cnt1
        elig = valid & (((kc >> jnp.uint32(16)) & jnp.uint32(0xFF)) == b2u)
        b = ((kc >> jnp.uint32(8)) & jnp.uint32(0xFF)).astype(jnp.int32)
        plsc.addupdate_scatter(hist, [lane256 + b], ones, mask=elig)

    b3, k, _ = _find_bucket(hist, tot, k)
    p23 = (b2u << jnp.uint32(8)) | b3.astype(jnp.uint32)
    zero_hist()

    @plsc.parallel_loop(0, nv1, unroll=4)
    def _(i):
        kc = lax.bitcast_convert_type(cand[pl.ds(i * 16, 16)], jnp.uint32)
        valid = (i * 16 + lane) < cnt1
        elig = valid & (((kc >> jnp.uint32(8)) & jnp.uint32(0xFFFF)) == p23)
        b = (kc & jnp.uint32(0xFF)).astype(jnp.int32)
        plsc.addupdate_scatter(hist, [lane256 + b], ones, mask=elig)

    b4, k, _ = _find_bucket(hist, tot, k)
    thresh = (
        (b1u << jnp.uint32(24))
        | (p23 << jnp.uint32(8))
        | b4.astype(jnp.uint32)
    )

    @plsc.parallel_loop(0, _NV, unroll=_UNROLL)
    def _(i):
        key = kv[pl.ds(i * 16, 16)]
        xc = xv[pl.ds(i * 16, 16)]
        xv[pl.ds(i * 16, 16)] = jnp.where(key >= thresh, xc, jnp.float32(0.0))

    pltpu.sync_copy(xv, out_hbm.at[pl.ds(base, _COLS)])


_sc_kernel = functools.partial(
    pl.kernel,
    out_type=jax.ShapeDtypeStruct((_SC_ROWS * _COLS,), jnp.float32),
    mesh=plsc.VectorSubcoreMesh(core_axis_name="c", subcore_axis_name="s"),
    scratch_types=[
        pltpu.VMEM((_COLS,), jnp.float32),
        pltpu.VMEM((_COLS,), jnp.uint32),
        pltpu.VMEM((_COLS,), jnp.int32),
        pltpu.VMEM((16 * 256,), jnp.int32),
        pltpu.VMEM((256,), jnp.int32),
    ],
    compiler_params=pltpu.CompilerParams(needs_layout_passes=False),
)(_sc_body)


def _tc_body(x_ref, o_ref):
    x = x_ref[...]
    u = lax.bitcast_convert_type(x, jnp.uint32)
    sign = u >> jnp.uint32(31)
    key = u ^ (jnp.uint32(0x80000000) + sign * jnp.uint32(0x7FFFFFFF))

    def step(i, p):
        bit = jnp.uint32(31) - i.astype(jnp.uint32)
        cand = p | (jnp.uint32(1) << bit)
        cnt = jnp.sum((key >= cand).astype(jnp.int32), axis=1, keepdims=True)
        return jnp.where(cnt >= _TOPK, cand, p)

    p0 = jnp.zeros((x.shape[0], 1), jnp.uint32)
    thresh = lax.fori_loop(0, 32, step, p0)
    o_ref[...] = jnp.where(key >= thresh, x, jnp.float32(0.0))


def _tc_kernel(x):
    return pl.pallas_call(
        _tc_body,
        out_shape=jax.ShapeDtypeStruct((_TC_ROWS, _COLS), jnp.float32),
        grid=(_TC_ROWS // _TC_BLOCK,),
        in_specs=[pl.BlockSpec((_TC_BLOCK, _COLS), lambda i: (i, 0))],
        out_specs=pl.BlockSpec((_TC_BLOCK, _COLS), lambda i: (i, 0)),
    )(x)


def kernel(x):
    tc_out = _tc_kernel(x[_SC_ROWS:])
    sc_out = _sc_kernel(x[:_SC_ROWS].reshape(-1)).reshape(_SC_ROWS, _COLS)
    return jnp.concatenate([sc_out, tc_out], axis=0)
